# baseline (speedup 1.0000x reference)
import functools
import numpy as np
import jax
import jax.numpy as jnp
from jax import lax
from jax.experimental import pallas as pl
from jax.experimental.pallas import tpu as pltpu
from jax.experimental.pallas import tpu_sc as plsc

_D = 128
_H = 8
_DH = 16
_MODES = 6
_EPS = 1e-5
_NC = 2
_NS = 16
_NW = _NC * _NS
_CH = 80


def _ln(p, x):
    m = x.mean(-1, keepdims=True)
    v = ((x - m) ** 2).mean(-1, keepdims=True)
    return (x - m) / jnp.sqrt(v + _EPS) * p["g"] + p["b"]


def _lin(p, x):
    return x @ p["w"] + p["b"]


def _mesh():
    return plsc.VectorSubcoreMesh(core_axis_name="c", subcore_axis_name="s")


def _wid():
    return lax.axis_index("s") * _NC + lax.axis_index("c")


def _sc_edge_gather(nf, src3, dst3):
    kch = src3.shape[1]
    epw = kch * _CH
    e = _NW * epw

    def body(nf_h, src_h, dst_h, sf_h, df_h, srcv, dstv, bs, bd, sem1, sem2):
        w = _wid()
        pltpu.sync_copy(src_h.at[w], srcv)
        pltpu.sync_copy(dst_h.at[w], dstv)

        def chunk(j, carry):
            cs = pltpu.async_copy(nf_h.at[srcv.at[j]], bs, sem1)
            cd = pltpu.async_copy(nf_h.at[dstv.at[j]], bd, sem2)
            cs.wait()
            cd.wait()
            r0 = w * epw + j * _CH
            pltpu.sync_copy(bs, sf_h.at[pl.ds(r0, _CH)])
            pltpu.sync_copy(bd, df_h.at[pl.ds(r0, _CH)])
            return carry

        lax.fori_loop(0, kch, chunk, 0)

    f = pl.kernel(
        body,
        out_type=[jax.ShapeDtypeStruct((e, 16), jnp.float32),
                  jax.ShapeDtypeStruct((e, 16), jnp.float32)],
        mesh=_mesh(),
        compiler_params=pltpu.CompilerParams(use_tc_tiling_on_sc=False, needs_layout_passes=False),
        scratch_types=[
            pltpu.VMEM((kch, _CH), jnp.int32),
            pltpu.VMEM((kch, _CH), jnp.int32),
            pltpu.VMEM((_CH, 16), jnp.float32),
            pltpu.VMEM((_CH, 16), jnp.float32),
            pltpu.SemaphoreType.DMA,
            pltpu.SemaphoreType.DMA,
        ],
    )
    return f(nf, src3, dst3)


def _sc_pass_a(qt, knt, keh, src3, dst3):
    kch = src3.shape[1]
    epw = kch * _CH
    e = _NW * epw

    def body(qt_h, knt_h, ke_h, src_h, dst_h, al_h, srcv, dstv, qv, knv, kev,
             av, sem1, sem2):
        w = _wid()
        pltpu.sync_copy(src_h.at[w], srcv)
        pltpu.sync_copy(dst_h.at[w], dstv)

        def chunk(j, carry):
            cq = pltpu.async_copy(qt_h.at[dstv.at[j]], qv, sem1)
            ck = pltpu.async_copy(knt_h.at[srcv.at[j]], knv, sem2)
            pltpu.sync_copy(ke_h.at[pl.ds(w * epw + j * _CH, _CH)], kev)
            cq.wait()
            ck.wait()

            lane0 = lax.iota(jnp.int32, 16) == 0

            def edge(ei, c2):
                arow = jnp.full((16,), lax.shift_right_logical(ei, 1), jnp.int32)
                acol = jnp.full((16,), lax.bitwise_and(ei, 1) * 8, jnp.int32)
                for h in range(_H):
                    sl = pl.ds(h * _DH, _DH)
                    s = jnp.sum(qv[ei, sl] * (knv[ei, sl] + kev[ei, sl]))
                    plsc.store_scatter(av, [arow, acol + h],
                                       jnp.full((16,), s * 0.25, jnp.float32),
                                       mask=lane0)
                return c2

            lax.fori_loop(0, _CH, edge, 0)
            pltpu.sync_copy(av, al_h.at[pl.ds((w * epw + j * _CH) // 2, _CH // 2)])
            return carry

        lax.fori_loop(0, kch, chunk, 0)

    f = pl.kernel(
        body,
        out_type=jax.ShapeDtypeStruct((e // 2, 16), jnp.float32),
        mesh=_mesh(),
        compiler_params=pltpu.CompilerParams(use_tc_tiling_on_sc=False, needs_layout_passes=False),
        scratch_types=[
            pltpu.VMEM((kch, _CH), jnp.int32),
            pltpu.VMEM((kch, _CH), jnp.int32),
            pltpu.VMEM((_CH, _D), jnp.float32),
            pltpu.VMEM((_CH, _D), jnp.float32),
            pltpu.VMEM((_CH, _D), jnp.float32),
            pltpu.VMEM((_CH // 2, 16), jnp.float32),
            pltpu.SemaphoreType.DMA,
            pltpu.SemaphoreType.DMA,
        ],
    )
    return f(qt, knt, keh, src3, dst3)


_CHB = 40


def _sc_pass_b(vt, veh, al2, gm16, zm, zd, src3, dst3, n):
    kch = src3.shape[1]
    epw = kch * _CHB
    nrs = n // _NS

    def body(vt_h, ve_h, al_h, gm_h, zm_h, zd_h, src_h, dst_h, om_h, od_h,
             srcv, dstv, vv, vev, av, wb, msgb, gmv, accm, accd, sem1):
        c = lax.axis_index("c")
        s = lax.axis_index("s")
        w = s * _NC + c
        pltpu.sync_copy(src_h.at[w], srcv)
        pltpu.sync_copy(dst_h.at[w], dstv)
        pltpu.sync_copy(gm_h, gmv)
        r0 = s * nrs
        pltpu.sync_copy(zm_h.at[pl.ds(r0, nrs)], accm.at[pl.ds(r0, nrs)])
        pltpu.sync_copy(zd_h.at[pl.ds(r0, nrs)], accd.at[pl.ds(r0, nrs)])

        def zrow(r, c2):
            wb[r, :] = jnp.zeros((16,), jnp.float32)
            return c2

        lax.fori_loop(0, _CHB, zrow, 0)
        plsc.subcore_barrier()

        i16 = lax.iota(jnp.int32, 16)
        half = lax.shift_right_logical(i16, 3)
        h8 = lax.bitwise_and(i16, 7)

        def chunk(j, carry):
            cv = pltpu.async_copy(vt_h.at[srcv.at[j]], vv, sem1)
            pltpu.sync_copy(ve_h.at[pl.ds(w * epw + j * _CHB, _CHB)], vev)
            pltpu.sync_copy(al_h.at[pl.ds((w * epw + j * _CHB) // 2, _CHB // 2)], av)
            gmr = gmv[...]

            def wrow(r, c2):
                ww = jnp.exp(av[r, :] - gmr)
                rows = r * 2 + half
                plsc.store_scatter(wb, [rows, h8], ww)
                return c2

            lax.fori_loop(0, _CHB // 2, wrow, 0)
            cv.wait()

            def edge(ei, c2):
                wv = wb[ei, :]
                for h in range(_H):
                    wsc = wv[h]
                    mv = (vv[ei, pl.ds(h * 16, 16)] +
                          vev[ei, pl.ds(h * 16, 16)]) * wsc
                    msgb[ei, pl.ds(h * 16, 16)] = mv
                return c2

            lax.fori_loop(0, _CHB, edge, 0)
            pltpu.sync_copy(msgb, accm.at[dstv.at[j]], add=True)
            pltpu.sync_copy(wb, accd.at[dstv.at[j]], add=True)
            return carry

        lax.fori_loop(0, kch, chunk, 0)
        plsc.subcore_barrier()
        pltpu.sync_copy(accm.at[pl.ds(r0, nrs)], om_h.at[c, pl.ds(r0, nrs)])
        pltpu.sync_copy(accd.at[pl.ds(r0, nrs)], od_h.at[c, pl.ds(r0, nrs)])

    f = pl.kernel(
        body,
        out_type=[jax.ShapeDtypeStruct((_NC, n, _D), jnp.float32),
                  jax.ShapeDtypeStruct((_NC, n, 16), jnp.float32)],
        mesh=_mesh(),
        compiler_params=pltpu.CompilerParams(use_tc_tiling_on_sc=False, needs_layout_passes=False),
        scratch_types=[
            pltpu.VMEM((kch, _CHB), jnp.int32),
            pltpu.VMEM((kch, _CHB), jnp.int32),
            pltpu.VMEM((_CHB, _D), jnp.float32),
            pltpu.VMEM((_CHB, _D), jnp.float32),
            pltpu.VMEM((_CHB // 2, 16), jnp.float32),
            pltpu.VMEM((_CHB, 16), jnp.float32),
            pltpu.VMEM((_CHB, _D), jnp.float32),
            pltpu.VMEM((16,), jnp.float32),
            pltpu.VMEM_SHARED((n, _D), jnp.float32),
            pltpu.VMEM_SHARED((n, 16), jnp.float32),
            pltpu.SemaphoreType.DMA,
        ],
    )
    return f(vt, veh, al2, gm16, zm, zd, src3, dst3)


def _final_stage_kernel(x_ref, g_ref, b_ref, w_ref, wb_ref, o_ref):
    x = x_ref[:]
    m = x.mean(-1, keepdims=True)
    v = ((x - m) ** 2).mean(-1, keepdims=True)
    xn = (x - m) * lax.rsqrt(v + _EPS) * g_ref[:] + b_ref[:]
    o_ref[:] = jnp.dot(xn, w_ref[:], preferred_element_type=jnp.float32) + wb_ref[:]


def _final_stage(x, norm, proj):
    n = x.shape[0]
    br = 1000
    return pl.pallas_call(
        _final_stage_kernel,
        grid=(n // br,),
        in_specs=[
            pl.BlockSpec((br, _D), lambda i: (i, 0)),
            pl.BlockSpec((_D,), lambda i: (0,)),
            pl.BlockSpec((_D,), lambda i: (0,)),
            pl.BlockSpec((_D, _MODES * _D), lambda i: (0, 0)),
            pl.BlockSpec((_MODES * _D,), lambda i: (0,)),
        ],
        out_specs=pl.BlockSpec((br, _MODES * _D), lambda i: (i, 0)),
        out_shape=jax.ShapeDtypeStruct((n, _MODES * _D), jnp.float32),
    )(x, norm["g"], norm["b"], proj["w"], proj["b"])


def _ln_block(x, g, b):
    m = x.mean(-1, keepdims=True)
    v = ((x - m) ** 2).mean(-1, keepdims=True)
    return (x - m) * lax.rsqrt(v + _EPS) * g + b


def _mm(x, w, b):
    return jnp.dot(x, w, preferred_element_type=jnp.float32) + b


def _edge_dense_kernel(sf_ref, df_ref, *refs):
    w10, b10, g10, bb10, w20, b20, w11, b11, g11, bb11, w21, b21, \
        ag1g, ag1b, agw, agb, ag2g, ag2b = refs[:18]
    wrefs = refs[18:24]
    brefs = refs[24:30]
    orefs = refs[30:]
    sf = sf_ref[:]
    df = df_ref[:]
    rel0 = sf[:, 0:1] - df[:, 0:1]
    rel1 = sf[:, 1:2] - df[:, 1:2]
    rp0 = rel0 * df[:, 4:5] + rel1 * df[:, 6:7]
    rp1 = rel0 * df[:, 5:6] + rel1 * df[:, 7:8]
    tf0 = sf[:, 2:3] * df[:, 2:3] + sf[:, 3:4] * df[:, 3:4]
    tf1 = sf[:, 3:4] * df[:, 2:3] - sf[:, 2:3] * df[:, 3:4]
    h0 = jax.nn.relu(_ln_block(rp0 * w10[0:1, :] + rp1 * w10[1:2, :] + b10[:],
                               g10[:], bb10[:]))
    h1 = jax.nn.relu(_ln_block(tf0 * w11[0:1, :] + tf1 * w11[1:2, :] + b11[:],
                               g11[:], bb11[:]))
    o = _mm(h0, w20[:], b20[:]) + _mm(h1, w21[:], b21[:])
    o = jax.nn.relu(_ln_block(o, ag1g[:], ag1b[:]))
    o = _mm(o, agw[:], agb[:])
    ea = _ln_block(o, ag2g[:], ag2b[:])
    for k in range(6):
        orefs[k][:] = _mm(ea, wrefs[k][:], brefs[k][:])


def _edge_dense(sf, df, rel_p, layers):
    e = sf.shape[0]
    be = 1000
    vecs = []
    mats = []

    def vspec():
        return pl.BlockSpec((_D,), lambda i: (0,))

    br0, br1 = rel_p["branches"]
    w10 = jnp.zeros((8, _D), jnp.float32).at[:2].set(br0["lin1"]["w"])
    w11 = jnp.zeros((8, _D), jnp.float32).at[:2].set(br1["lin1"]["w"])
    args = [w10, br0["lin1"]["b"], br0["ln"]["g"], br0["ln"]["b"],
            br0["lin2"]["w"], br0["lin2"]["b"],
            w11, br1["lin1"]["b"], br1["ln"]["g"], br1["ln"]["b"],
            br1["lin2"]["w"], br1["lin2"]["b"],
            rel_p["aggr_ln1"]["g"], rel_p["aggr_ln1"]["b"],
            rel_p["aggr_lin"]["w"], rel_p["aggr_lin"]["b"],
            rel_p["aggr_ln2"]["g"], rel_p["aggr_ln2"]["b"]]
    args += [lp[k]["w"] for lp in layers for k in ("lin_k_edge", "lin_v_edge")]
    args += [lp[k]["b"] for lp in layers for k in ("lin_k_edge", "lin_v_edge")]
    wspecs = []
    for a in args:
        if a.ndim == 1:
            wspecs.append(vspec())
        else:
            sh = a.shape
            wspecs.append(pl.BlockSpec(sh, lambda i: (0, 0)))
    outs = pl.pallas_call(
        _edge_dense_kernel,
        grid=(e // be,),
        in_specs=[pl.BlockSpec((be, 16), lambda i: (i, 0)),
                  pl.BlockSpec((be, 16), lambda i: (i, 0))] + wspecs,
        out_specs=[pl.BlockSpec((be, _D), lambda i: (i, 0))] * 6,
        out_shape=[jax.ShapeDtypeStruct((e, _D), jnp.float32)] * 6,
    )(sf, df, *args)
    return outs


def _node_pre_kernel(x_ref, g_ref, b_ref, wq, bq, wk, bk, wv, bv,
                     h_ref, q_ref, k_ref, v_ref):
    h = _ln_block(x_ref[:], g_ref[:], b_ref[:])
    h_ref[:] = h
    q_ref[:] = _mm(h, wq[:], bq[:])
    k_ref[:] = _mm(h, wk[:], bk[:])
    v_ref[:] = _mm(h, wv[:], bv[:])


def _node_pre(x, lp):
    n = x.shape[0]
    bn = 1000
    mspec = pl.BlockSpec((_D, _D), lambda i: (0, 0))
    vspec = pl.BlockSpec((_D,), lambda i: (0,))
    rspec = pl.BlockSpec((bn, _D), lambda i: (i, 0))
    return pl.pallas_call(
        _node_pre_kernel,
        grid=(n // bn,),
        in_specs=[rspec, vspec, vspec, mspec, vspec, mspec, vspec, mspec, vspec],
        out_specs=[rspec] * 4,
        out_shape=[jax.ShapeDtypeStruct((n, _D), jnp.float32)] * 4,
    )(x, lp["norm1"]["g"], lp["norm1"]["b"],
      lp["lin_q_node"]["w"], lp["lin_q_node"]["b"],
      lp["lin_k_node"]["w"], lp["lin_k_node"]["b"],
      lp["lin_v_node"]["w"], lp["lin_v_node"]["b"])


def _node_post_kernel(om0, om1, od0, od1, rmat, h_ref, x_ref,
                      wih, bih, whh, bhh, wsf, bsf, wop, bop,
                      n2g, n2b, w1, b1, w2, b2, o_ref):
    den = jnp.dot(od0[:] + od1[:], rmat[:],
                  preferred_element_type=jnp.float32) + 1e-16
    agg = (om0[:] + om1[:]) / den
    h = h_ref[:]
    gate = jax.nn.sigmoid(_mm(agg, wih[:], bih[:]) + _mm(h, whh[:], bhh[:]))
    upd = agg + gate * (_mm(h, wsf[:], bsf[:]) - agg)
    x2 = x_ref[:] + _mm(upd, wop[:], bop[:])
    h2 = _ln_block(x2, n2g[:], n2b[:])
    ff = _mm(jax.nn.relu(_mm(h2, w1[:], b1[:])), w2[:], b2[:])
    o_ref[:] = x2 + ff


def _node_post(om, od, rmat, h, x, lp):
    n = x.shape[0]
    bn = 1000
    rspec = pl.BlockSpec((bn, _D), lambda i: (i, 0))
    dspec = pl.BlockSpec((bn, 16), lambda i: (i, 0))
    vspec = pl.BlockSpec((_D,), lambda i: (0,))

    def mspec(a, b):
        return pl.BlockSpec((a, b), lambda i: (0, 0))

    return pl.pallas_call(
        _node_post_kernel,
        grid=(n // bn,),
        in_specs=[rspec, rspec, dspec, dspec, mspec(16, _D), rspec, rspec,
                  mspec(_D, _D), vspec, mspec(_D, _D), vspec,
                  mspec(_D, _D), vspec, mspec(_D, _D), vspec,
                  vspec, vspec,
                  mspec(_D, 4 * _D), pl.BlockSpec((4 * _D,), lambda i: (0,)),
                  mspec(4 * _D, _D), vspec],
        out_specs=rspec,
        out_shape=jax.ShapeDtypeStruct((n, _D), jnp.float32),
    )(om[0], om[1], od[0], od[1], rmat, h, x,
      lp["lin_ih"]["w"], lp["lin_ih"]["b"],
      lp["lin_hh"]["w"], lp["lin_hh"]["b"],
      lp["lin_self"]["w"], lp["lin_self"]["b"],
      lp["out_proj"]["w"], lp["out_proj"]["b"],
      lp["norm2"]["g"], lp["norm2"]["b"],
      lp["mlp_lin1"]["w"], lp["mlp_lin1"]["b"],
      lp["mlp_lin2"]["w"], lp["mlp_lin2"]["b"])


def _alpha_max_kernel(a_ref, o_ref):
    @pl.when(pl.program_id(0) == 0)
    def _init():
        o_ref[:] = jnp.full((8, 16), -jnp.inf, jnp.float32)

    blk = a_ref[:].reshape(-1, 8, 16).max(axis=0)
    o_ref[:] = jnp.maximum(o_ref[:], blk)


def _alpha_max(al2):
    m = al2.shape[0]
    bm = 1000
    out = pl.pallas_call(
        _alpha_max_kernel,
        grid=(m // bm,),
        in_specs=[pl.BlockSpec((bm, 16), lambda i: (i, 0))],
        out_specs=pl.BlockSpec((8, 16), lambda i: (0, 0)),
        out_shape=jax.ShapeDtypeStruct((8, 16), jnp.float32),
    )(al2)
    return out.max(axis=0)


def _layer(lp, x, src3, dst3, src3b, dst3b, ke, ve, rmat, zm, zd, n):
    h, q, kn, vn = _node_pre(x, lp)
    al2 = _sc_pass_a(q, kn, ke, src3, dst3)
    m16 = _alpha_max(al2)
    gm8 = jnp.maximum(m16[:8], m16[8:])
    gm16 = jnp.concatenate([gm8, gm8])
    om, od = _sc_pass_b(vn, ve, al2, gm16, zm, zd, src3b, dst3b, n)
    return _node_post(om, od, rmat, h, x, lp)


@jax.jit
def _run(local_embed, edge_index, positions, rotate_mat, rotate_angles, params):
    n, d = local_embed.shape
    t = positions.shape[1]
    e = edge_index.shape[1]
    kch = e // (_NW * _CH)
    src3 = edge_index[0].reshape(_NW, kch, _CH)
    dst3 = edge_index[1].reshape(_NW, kch, _CH)
    kchb = e // (_NW * _CHB)
    src3b = edge_index[0].reshape(_NW, kchb, _CHB)
    dst3b = edge_index[1].reshape(_NW, kchb, _CHB)

    p19 = positions[:, t - 1]
    ca = jnp.cos(rotate_angles)[:, None]
    sa = jnp.sin(rotate_angles)[:, None]
    nf = jnp.concatenate(
        [p19, ca, sa, rotate_mat.reshape(n, 4), jnp.zeros((n, 8), jnp.float32)],
        axis=1)
    sf, df = _sc_edge_gather(nf, src3, dst3)

    keve = _edge_dense(sf, df, params["rel_embed"], params["layers"])

    rmat = jnp.concatenate(
        [jnp.repeat(jnp.eye(_H, dtype=jnp.float32), _DH, axis=1),
         jnp.zeros((_H, _D), jnp.float32)], axis=0)
    zm = jnp.zeros((n, _D), jnp.float32)
    zd = jnp.zeros((n, 16), jnp.float32)
    x = local_embed
    for li, lp in enumerate(params["layers"]):
        x = _layer(lp, x, src3, dst3, src3b, dst3b, keve[2 * li],
                   keve[2 * li + 1], rmat, zm, zd, n)
    out = _final_stage(x, params["norm"], params["multihead_proj"])
    return jnp.transpose(out.reshape(n, _MODES, d), (1, 0, 2))


def kernel(local_embed, edge_index, positions, rotate_mat, rotate_angles, padding_mask, params):
    return _run(local_embed, edge_index, positions, rotate_mat, rotate_angles, params)


# speedup vs baseline: 1.3733x; 1.3733x over previous
import functools
import numpy as np
import jax
import jax.numpy as jnp
from jax import lax
from jax.experimental import pallas as pl
from jax.experimental.pallas import tpu as pltpu
from jax.experimental.pallas import tpu_sc as plsc

_D = 128
_H = 8
_DH = 16
_MODES = 6
_EPS = 1e-5
_DP = 144
_NC = 2
_NS = 16
_NW = _NC * _NS
_CH = 80


def _ln(p, x):
    m = x.mean(-1, keepdims=True)
    v = ((x - m) ** 2).mean(-1, keepdims=True)
    return (x - m) / jnp.sqrt(v + _EPS) * p["g"] + p["b"]


def _lin(p, x):
    return x @ p["w"] + p["b"]


def _mesh():
    return plsc.VectorSubcoreMesh(core_axis_name="c", subcore_axis_name="s")


def _wid():
    return lax.axis_index("s") * _NC + lax.axis_index("c")


def _sc_edge_gather(nf, src3, dst3):
    kch = src3.shape[1]
    epw = kch * _CH
    e = _NW * epw

    def body(nf_h, src_h, dst_h, sf_h, df_h, srcv, dstv, bs, bd, sem1, sem2):
        w = _wid()
        pltpu.sync_copy(src_h.at[w], srcv)
        pltpu.sync_copy(dst_h.at[w], dstv)

        def chunk(j, carry):
            cs = pltpu.async_copy(nf_h.at[srcv.at[j]], bs, sem1)
            cd = pltpu.async_copy(nf_h.at[dstv.at[j]], bd, sem2)
            cs.wait()
            cd.wait()
            r0 = w * epw + j * _CH
            pltpu.sync_copy(bs, sf_h.at[pl.ds(r0, _CH)])
            pltpu.sync_copy(bd, df_h.at[pl.ds(r0, _CH)])
            return carry

        lax.fori_loop(0, kch, chunk, 0)

    f = pl.kernel(
        body,
        out_type=[jax.ShapeDtypeStruct((e, 16), jnp.float32),
                  jax.ShapeDtypeStruct((e, 16), jnp.float32)],
        mesh=_mesh(),
        compiler_params=pltpu.CompilerParams(use_tc_tiling_on_sc=False, needs_layout_passes=False),
        scratch_types=[
            pltpu.VMEM((kch, _CH), jnp.int32),
            pltpu.VMEM((kch, _CH), jnp.int32),
            pltpu.VMEM((_CH, 16), jnp.float32),
            pltpu.VMEM((_CH, 16), jnp.float32),
            pltpu.SemaphoreType.DMA,
            pltpu.SemaphoreType.DMA,
        ],
    )
    return f(nf, src3, dst3)


def _sc_pass_a(qt, knt, keh, src3, dst3):
    kch = src3.shape[1]
    epw = kch * _CH
    e = _NW * epw

    def body(qt_h, knt_h, ke_h, src_h, dst_h, al_h, srcv, dstv, qv, knv, kev,
             av, sem1, sem2):
        w = _wid()
        pltpu.sync_copy(src_h.at[w], srcv)
        pltpu.sync_copy(dst_h.at[w], dstv)

        def chunk(j, carry):
            cq = pltpu.async_copy(qt_h.at[dstv.at[j]], qv, sem1)
            ck = pltpu.async_copy(knt_h.at[srcv.at[j]], knv, sem2)
            pltpu.sync_copy(ke_h.at[pl.ds(w * epw + j * _CH, _CH)], kev)
            cq.wait()
            ck.wait()

            def group(g, c2):
                erow = lax.iota(jnp.int32, 16) + g * 16
                arow = lax.shift_right_logical(erow, 1)
                apar = lax.bitwise_and(erow, 1) * 8

                def head(h, c3):
                    col0 = h * 16
                    acc = jnp.zeros((16,), jnp.float32)
                    for d in range(_DH):
                        ccol = jnp.full((16,), col0 + d, jnp.int32)
                        qq = plsc.load_gather(qv, [erow, ccol])
                        kk = (plsc.load_gather(knv, [erow, ccol]) +
                              plsc.load_gather(kev, [erow, ccol]))
                        acc = acc + qq * kk
                    plsc.store_scatter(av, [arow, apar + h], acc * 0.25)
                    return c3

                lax.fori_loop(0, _H, head, c2)
                return c2

            lax.fori_loop(0, _CH // 16, group, 0)
            pltpu.sync_copy(av, al_h.at[pl.ds((w * epw + j * _CH) // 2, _CH // 2)])
            return carry

        lax.fori_loop(0, kch, chunk, 0)

    f = pl.kernel(
        body,
        out_type=jax.ShapeDtypeStruct((e // 2, 16), jnp.float32),
        mesh=_mesh(),
        compiler_params=pltpu.CompilerParams(use_tc_tiling_on_sc=False, needs_layout_passes=False),
        scratch_types=[
            pltpu.VMEM((kch, _CH), jnp.int32),
            pltpu.VMEM((kch, _CH), jnp.int32),
            pltpu.VMEM((_CH, _DP), jnp.float32),
            pltpu.VMEM((_CH, _DP), jnp.float32),
            pltpu.VMEM((_CH, _DP), jnp.float32),
            pltpu.VMEM((_CH // 2, 16), jnp.float32),
            pltpu.SemaphoreType.DMA,
            pltpu.SemaphoreType.DMA,
        ],
    )
    return f(qt, knt, keh, src3, dst3)


_CHB = 40


def _sc_pass_b(vt, veh, al2, gm16, zm, zd, src3, dst3, n):
    kch = src3.shape[1]
    epw = kch * _CHB
    nrs = n // _NS

    def body(vt_h, ve_h, al_h, gm_h, zm_h, zd_h, src_h, dst_h, om_h, od_h,
             srcv, dstv, vv, vev, av, wb, msgb, gmv, accm, accd, sem1):
        c = lax.axis_index("c")
        s = lax.axis_index("s")
        w = s * _NC + c
        pltpu.sync_copy(src_h.at[w], srcv)
        pltpu.sync_copy(dst_h.at[w], dstv)
        pltpu.sync_copy(gm_h, gmv)
        r0 = s * nrs
        pltpu.sync_copy(zm_h.at[pl.ds(r0, nrs)], accm.at[pl.ds(r0, nrs)])
        pltpu.sync_copy(zd_h.at[pl.ds(r0, nrs)], accd.at[pl.ds(r0, nrs)])

        def zrow(r, c2):
            wb[r, :] = jnp.zeros((16,), jnp.float32)
            return c2

        lax.fori_loop(0, _CHB, zrow, 0)
        plsc.subcore_barrier()

        i16 = lax.iota(jnp.int32, 16)
        half = lax.shift_right_logical(i16, 3)
        h8 = lax.bitwise_and(i16, 7)

        def chunk(j, carry):
            cv = pltpu.async_copy(vt_h.at[srcv.at[j]], vv, sem1)
            pltpu.sync_copy(ve_h.at[pl.ds(w * epw + j * _CHB, _CHB)], vev)
            pltpu.sync_copy(al_h.at[pl.ds((w * epw + j * _CHB) // 2, _CHB // 2)], av)
            gmr = gmv[...]

            def wrow(r, c2):
                ww = jnp.exp(av[r, :] - gmr)
                rows = r * 2 + half
                plsc.store_scatter(wb, [rows, h8], ww)
                return c2

            lax.fori_loop(0, _CHB // 2, wrow, 0)
            cv.wait()

            def edge(ei, c2):
                wv = wb[ei, :]
                for h in range(_H):
                    wsc = wv[h]
                    mv = (vv[ei, pl.ds(h * 16, 16)] +
                          vev[ei, pl.ds(h * 16, 16)]) * wsc
                    msgb[ei, pl.ds(h * 16, 16)] = mv
                return c2

            lax.fori_loop(0, _CHB, edge, 0)
            pltpu.sync_copy(msgb, accm.at[dstv.at[j]], add=True)
            pltpu.sync_copy(wb, accd.at[dstv.at[j]], add=True)
            return carry

        lax.fori_loop(0, kch, chunk, 0)
        plsc.subcore_barrier()
        pltpu.sync_copy(accm.at[pl.ds(r0, nrs)], om_h.at[c, pl.ds(r0, nrs)])
        pltpu.sync_copy(accd.at[pl.ds(r0, nrs)], od_h.at[c, pl.ds(r0, nrs)])

    f = pl.kernel(
        body,
        out_type=[jax.ShapeDtypeStruct((_NC, n, _D), jnp.float32),
                  jax.ShapeDtypeStruct((_NC, n, 16), jnp.float32)],
        mesh=_mesh(),
        compiler_params=pltpu.CompilerParams(use_tc_tiling_on_sc=False, needs_layout_passes=False),
        scratch_types=[
            pltpu.VMEM((kch, _CHB), jnp.int32),
            pltpu.VMEM((kch, _CHB), jnp.int32),
            pltpu.VMEM((_CHB, _D), jnp.float32),
            pltpu.VMEM((_CHB, _D), jnp.float32),
            pltpu.VMEM((_CHB // 2, 16), jnp.float32),
            pltpu.VMEM((_CHB, 16), jnp.float32),
            pltpu.VMEM((_CHB, _D), jnp.float32),
            pltpu.VMEM((16,), jnp.float32),
            pltpu.VMEM_SHARED((n, _D), jnp.float32),
            pltpu.VMEM_SHARED((n, 16), jnp.float32),
            pltpu.SemaphoreType.DMA,
        ],
    )
    return f(vt, veh, al2, gm16, zm, zd, src3, dst3)


def _final_stage_kernel(x_ref, g_ref, b_ref, w_ref, wb_ref, o_ref):
    x = x_ref[:]
    m = x.mean(-1, keepdims=True)
    v = ((x - m) ** 2).mean(-1, keepdims=True)
    xn = (x - m) * lax.rsqrt(v + _EPS) * g_ref[:] + b_ref[:]
    o_ref[:] = jnp.dot(xn, w_ref[:], preferred_element_type=jnp.float32) + wb_ref[:]


def _final_stage(x, norm, proj):
    n = x.shape[0]
    br = 1000
    return pl.pallas_call(
        _final_stage_kernel,
        grid=(n // br,),
        in_specs=[
            pl.BlockSpec((br, _D), lambda i: (i, 0)),
            pl.BlockSpec((_D,), lambda i: (0,)),
            pl.BlockSpec((_D,), lambda i: (0,)),
            pl.BlockSpec((_D, _MODES * _D), lambda i: (0, 0)),
            pl.BlockSpec((_MODES * _D,), lambda i: (0,)),
        ],
        out_specs=pl.BlockSpec((br, _MODES * _D), lambda i: (i, 0)),
        out_shape=jax.ShapeDtypeStruct((n, _MODES * _D), jnp.float32),
    )(x, norm["g"], norm["b"], proj["w"], proj["b"])


def _ln_block(x, g, b):
    m = x.mean(-1, keepdims=True)
    v = ((x - m) ** 2).mean(-1, keepdims=True)
    return (x - m) * lax.rsqrt(v + _EPS) * g + b


def _mm(x, w, b):
    return jnp.dot(x, w, preferred_element_type=jnp.float32) + b


def _edge_dense_kernel(sf_ref, df_ref, *refs):
    w10, b10, g10, bb10, w20, b20, w11, b11, g11, bb11, w21, b21, \
        ag1g, ag1b, agw, agb, ag2g, ag2b = refs[:18]
    wrefs = refs[18:24]
    brefs = refs[24:30]
    orefs = refs[30:]
    sf = sf_ref[:]
    df = df_ref[:]
    rel0 = sf[:, 0:1] - df[:, 0:1]
    rel1 = sf[:, 1:2] - df[:, 1:2]
    rp0 = rel0 * df[:, 4:5] + rel1 * df[:, 6:7]
    rp1 = rel0 * df[:, 5:6] + rel1 * df[:, 7:8]
    tf0 = sf[:, 2:3] * df[:, 2:3] + sf[:, 3:4] * df[:, 3:4]
    tf1 = sf[:, 3:4] * df[:, 2:3] - sf[:, 2:3] * df[:, 3:4]
    h0 = jax.nn.relu(_ln_block(rp0 * w10[0:1, :] + rp1 * w10[1:2, :] + b10[:],
                               g10[:], bb10[:]))
    h1 = jax.nn.relu(_ln_block(tf0 * w11[0:1, :] + tf1 * w11[1:2, :] + b11[:],
                               g11[:], bb11[:]))
    o = _mm(h0, w20[:], b20[:]) + _mm(h1, w21[:], b21[:])
    o = jax.nn.relu(_ln_block(o, ag1g[:], ag1b[:]))
    o = _mm(o, agw[:], agb[:])
    ea = _ln_block(o, ag2g[:], ag2b[:])
    zpad = jnp.zeros((ea.shape[0], _DP - _D), jnp.float32)
    for k in range(6):
        o = _mm(ea, wrefs[k][:], brefs[k][:])
        if k % 2 == 0:
            orefs[k][:, :_D] = o
            orefs[k][:, _D:] = zpad
        else:
            orefs[k][:] = o


def _edge_dense(sf, df, rel_p, layers):
    e = sf.shape[0]
    be = 1000
    vecs = []
    mats = []

    def vspec():
        return pl.BlockSpec((_D,), lambda i: (0,))

    br0, br1 = rel_p["branches"]
    w10 = jnp.zeros((8, _D), jnp.float32).at[:2].set(br0["lin1"]["w"])
    w11 = jnp.zeros((8, _D), jnp.float32).at[:2].set(br1["lin1"]["w"])
    args = [w10, br0["lin1"]["b"], br0["ln"]["g"], br0["ln"]["b"],
            br0["lin2"]["w"], br0["lin2"]["b"],
            w11, br1["lin1"]["b"], br1["ln"]["g"], br1["ln"]["b"],
            br1["lin2"]["w"], br1["lin2"]["b"],
            rel_p["aggr_ln1"]["g"], rel_p["aggr_ln1"]["b"],
            rel_p["aggr_lin"]["w"], rel_p["aggr_lin"]["b"],
            rel_p["aggr_ln2"]["g"], rel_p["aggr_ln2"]["b"]]
    args += [lp[k]["w"] for lp in layers for k in ("lin_k_edge", "lin_v_edge")]
    args += [lp[k]["b"] for lp in layers for k in ("lin_k_edge", "lin_v_edge")]
    wspecs = []
    for a in args:
        if a.ndim == 1:
            wspecs.append(vspec())
        else:
            sh = a.shape
            wspecs.append(pl.BlockSpec(sh, lambda i: (0, 0)))
    outs = pl.pallas_call(
        _edge_dense_kernel,
        grid=(e // be,),
        in_specs=[pl.BlockSpec((be, 16), lambda i: (i, 0)),
                  pl.BlockSpec((be, 16), lambda i: (i, 0))] + wspecs,
        out_specs=[pl.BlockSpec((be, _DP if k % 2 == 0 else _D),
                                lambda i: (i, 0)) for k in range(6)],
        out_shape=[jax.ShapeDtypeStruct((e, _DP if k % 2 == 0 else _D),
                                        jnp.float32) for k in range(6)],
    )(sf, df, *args)
    return outs


def _node_pre_kernel(x_ref, g_ref, b_ref, wq, bq, wk, bk, wv, bv,
                     h_ref, q_ref, k_ref, v_ref):
    h = _ln_block(x_ref[:], g_ref[:], b_ref[:])
    h_ref[:] = h
    zpad = jnp.zeros((h.shape[0], _DP - _D), jnp.float32)
    q_ref[:, :_D] = _mm(h, wq[:], bq[:])
    q_ref[:, _D:] = zpad
    k_ref[:, :_D] = _mm(h, wk[:], bk[:])
    k_ref[:, _D:] = zpad
    v_ref[:] = _mm(h, wv[:], bv[:])


def _node_pre(x, lp):
    n = x.shape[0]
    bn = 1000
    mspec = pl.BlockSpec((_D, _D), lambda i: (0, 0))
    vspec = pl.BlockSpec((_D,), lambda i: (0,))
    rspec = pl.BlockSpec((bn, _D), lambda i: (i, 0))
    return pl.pallas_call(
        _node_pre_kernel,
        grid=(n // bn,),
        in_specs=[rspec, vspec, vspec, mspec, vspec, mspec, vspec, mspec, vspec],
        out_specs=[rspec,
                   pl.BlockSpec((bn, _DP), lambda i: (i, 0)),
                   pl.BlockSpec((bn, _DP), lambda i: (i, 0)),
                   rspec],
        out_shape=[jax.ShapeDtypeStruct((n, _D), jnp.float32),
                   jax.ShapeDtypeStruct((n, _DP), jnp.float32),
                   jax.ShapeDtypeStruct((n, _DP), jnp.float32),
                   jax.ShapeDtypeStruct((n, _D), jnp.float32)],
    )(x, lp["norm1"]["g"], lp["norm1"]["b"],
      lp["lin_q_node"]["w"], lp["lin_q_node"]["b"],
      lp["lin_k_node"]["w"], lp["lin_k_node"]["b"],
      lp["lin_v_node"]["w"], lp["lin_v_node"]["b"])


def _node_post_kernel(om0, om1, od0, od1, rmat, h_ref, x_ref,
                      wih, bih, whh, bhh, wsf, bsf, wop, bop,
                      n2g, n2b, w1, b1, w2, b2, o_ref):
    den = jnp.dot(od0[:] + od1[:], rmat[:],
                  preferred_element_type=jnp.float32) + 1e-16
    agg = (om0[:] + om1[:]) / den
    h = h_ref[:]
    gate = jax.nn.sigmoid(_mm(agg, wih[:], bih[:]) + _mm(h, whh[:], bhh[:]))
    upd = agg + gate * (_mm(h, wsf[:], bsf[:]) - agg)
    x2 = x_ref[:] + _mm(upd, wop[:], bop[:])
    h2 = _ln_block(x2, n2g[:], n2b[:])
    ff = _mm(jax.nn.relu(_mm(h2, w1[:], b1[:])), w2[:], b2[:])
    o_ref[:] = x2 + ff


def _node_post(om, od, rmat, h, x, lp):
    n = x.shape[0]
    bn = 1000
    rspec = pl.BlockSpec((bn, _D), lambda i: (i, 0))
    dspec = pl.BlockSpec((bn, 16), lambda i: (i, 0))
    vspec = pl.BlockSpec((_D,), lambda i: (0,))

    def mspec(a, b):
        return pl.BlockSpec((a, b), lambda i: (0, 0))

    return pl.pallas_call(
        _node_post_kernel,
        grid=(n // bn,),
        in_specs=[rspec, rspec, dspec, dspec, mspec(16, _D), rspec, rspec,
                  mspec(_D, _D), vspec, mspec(_D, _D), vspec,
                  mspec(_D, _D), vspec, mspec(_D, _D), vspec,
                  vspec, vspec,
                  mspec(_D, 4 * _D), pl.BlockSpec((4 * _D,), lambda i: (0,)),
                  mspec(4 * _D, _D), vspec],
        out_specs=rspec,
        out_shape=jax.ShapeDtypeStruct((n, _D), jnp.float32),
    )(om[0], om[1], od[0], od[1], rmat, h, x,
      lp["lin_ih"]["w"], lp["lin_ih"]["b"],
      lp["lin_hh"]["w"], lp["lin_hh"]["b"],
      lp["lin_self"]["w"], lp["lin_self"]["b"],
      lp["out_proj"]["w"], lp["out_proj"]["b"],
      lp["norm2"]["g"], lp["norm2"]["b"],
      lp["mlp_lin1"]["w"], lp["mlp_lin1"]["b"],
      lp["mlp_lin2"]["w"], lp["mlp_lin2"]["b"])


def _alpha_max_kernel(a_ref, o_ref):
    @pl.when(pl.program_id(0) == 0)
    def _init():
        o_ref[:] = jnp.full((8, 16), -jnp.inf, jnp.float32)

    blk = a_ref[:].reshape(-1, 8, 16).max(axis=0)
    o_ref[:] = jnp.maximum(o_ref[:], blk)


def _alpha_max(al2):
    m = al2.shape[0]
    bm = 1000
    out = pl.pallas_call(
        _alpha_max_kernel,
        grid=(m // bm,),
        in_specs=[pl.BlockSpec((bm, 16), lambda i: (i, 0))],
        out_specs=pl.BlockSpec((8, 16), lambda i: (0, 0)),
        out_shape=jax.ShapeDtypeStruct((8, 16), jnp.float32),
    )(al2)
    return out.max(axis=0)


def _layer(lp, x, src3, dst3, src3b, dst3b, ke, ve, rmat, zm, zd, n):
    h, q, kn, vn = _node_pre(x, lp)
    al2 = _sc_pass_a(q, kn, ke, src3, dst3)
    m16 = _alpha_max(al2)
    gm8 = jnp.maximum(m16[:8], m16[8:])
    gm16 = jnp.concatenate([gm8, gm8])
    om, od = _sc_pass_b(vn, ve, al2, gm16, zm, zd, src3b, dst3b, n)
    return _node_post(om, od, rmat, h, x, lp)


@jax.jit
def _run(local_embed, edge_index, positions, rotate_mat, rotate_angles, params):
    n, d = local_embed.shape
    t = positions.shape[1]
    e = edge_index.shape[1]
    kch = e // (_NW * _CH)
    src3 = edge_index[0].reshape(_NW, kch, _CH)
    dst3 = edge_index[1].reshape(_NW, kch, _CH)
    kchb = e // (_NW * _CHB)
    src3b = edge_index[0].reshape(_NW, kchb, _CHB)
    dst3b = edge_index[1].reshape(_NW, kchb, _CHB)

    p19 = positions[:, t - 1]
    ca = jnp.cos(rotate_angles)[:, None]
    sa = jnp.sin(rotate_angles)[:, None]
    nf = jnp.concatenate(
        [p19, ca, sa, rotate_mat.reshape(n, 4), jnp.zeros((n, 8), jnp.float32)],
        axis=1)
    sf, df = _sc_edge_gather(nf, src3, dst3)

    keve = _edge_dense(sf, df, params["rel_embed"], params["layers"])

    rmat = jnp.concatenate(
        [jnp.repeat(jnp.eye(_H, dtype=jnp.float32), _DH, axis=1),
         jnp.zeros((_H, _D), jnp.float32)], axis=0)
    zm = jnp.zeros((n, _D), jnp.float32)
    zd = jnp.zeros((n, 16), jnp.float32)
    x = local_embed
    for li, lp in enumerate(params["layers"]):
        x = _layer(lp, x, src3, dst3, src3b, dst3b, keve[2 * li],
                   keve[2 * li + 1], rmat, zm, zd, n)
    out = _final_stage(x, params["norm"], params["multihead_proj"])
    return jnp.transpose(out.reshape(n, _MODES, d), (1, 0, 2))


def kernel(local_embed, edge_index, positions, rotate_mat, rotate_angles, padding_mask, params):
    return _run(local_embed, edge_index, positions, rotate_mat, rotate_angles, params)


# speedup vs baseline: 1.5160x; 1.1039x over previous
import functools
import numpy as np
import jax
import jax.numpy as jnp
from jax import lax
from jax.experimental import pallas as pl
from jax.experimental.pallas import tpu as pltpu
from jax.experimental.pallas import tpu_sc as plsc

_D = 128
_H = 8
_DH = 16
_MODES = 6
_EPS = 1e-5
_DP = 144
_NC = 2
_NS = 16
_NW = _NC * _NS
_CH = 80


def _ln(p, x):
    m = x.mean(-1, keepdims=True)
    v = ((x - m) ** 2).mean(-1, keepdims=True)
    return (x - m) / jnp.sqrt(v + _EPS) * p["g"] + p["b"]


def _lin(p, x):
    return x @ p["w"] + p["b"]


def _mesh():
    return plsc.VectorSubcoreMesh(core_axis_name="c", subcore_axis_name="s")


def _wid():
    return lax.axis_index("s") * _NC + lax.axis_index("c")


def _sc_edge_gather(nf, src3, dst3):
    kch = src3.shape[1]
    epw = kch * _CH
    e = _NW * epw

    def body(nf_h, src_h, dst_h, sf_h, df_h, srcv, dstv, bs, bd, sem1, sem2):
        w = _wid()
        pltpu.sync_copy(src_h.at[w], srcv)
        pltpu.sync_copy(dst_h.at[w], dstv)

        def chunk(j, carry):
            cs = pltpu.async_copy(nf_h.at[srcv.at[j]], bs, sem1)
            cd = pltpu.async_copy(nf_h.at[dstv.at[j]], bd, sem2)
            cs.wait()
            cd.wait()
            r0 = w * epw + j * _CH
            pltpu.sync_copy(bs, sf_h.at[pl.ds(r0, _CH)])
            pltpu.sync_copy(bd, df_h.at[pl.ds(r0, _CH)])
            return carry

        lax.fori_loop(0, kch, chunk, 0)

    f = pl.kernel(
        body,
        out_type=[jax.ShapeDtypeStruct((e, 16), jnp.float32),
                  jax.ShapeDtypeStruct((e, 16), jnp.float32)],
        mesh=_mesh(),
        compiler_params=pltpu.CompilerParams(use_tc_tiling_on_sc=False, needs_layout_passes=False),
        scratch_types=[
            pltpu.VMEM((kch, _CH), jnp.int32),
            pltpu.VMEM((kch, _CH), jnp.int32),
            pltpu.VMEM((_CH, 16), jnp.float32),
            pltpu.VMEM((_CH, 16), jnp.float32),
            pltpu.SemaphoreType.DMA,
            pltpu.SemaphoreType.DMA,
        ],
    )
    return f(nf, src3, dst3)


def _sc_pass_a(qt, knt, keh, src3, dst3):
    kch = src3.shape[1]
    epw = kch * _CH
    e = _NW * epw

    def body(qt_h, knt_h, ke_h, src_h, dst_h, al_h, srcv, dstv,
             qv0, knv0, kev0, qv1, knv1, kev1, av, sem0, sem1):
        w = _wid()
        pltpu.sync_copy(src_h.at[w], srcv)
        pltpu.sync_copy(dst_h.at[w], dstv)
        bufs = ((qv0, knv0, kev0, sem0), (qv1, knv1, kev1, sem1))

        def issue(j, qv, knv, kev, sem):
            pltpu.async_copy(qt_h.at[dstv.at[j]], qv, sem)
            pltpu.async_copy(knt_h.at[srcv.at[j]], knv, sem)
            pltpu.async_copy(ke_h.at[pl.ds(w * epw + j * _CH, _CH)], kev, sem)

        def drain(qv, knv, kev, sem):
            pltpu.make_async_copy(qt_h.at[pl.ds(0, _CH)], qv, sem).wait()
            pltpu.make_async_copy(qt_h.at[pl.ds(0, _CH)], knv, sem).wait()
            pltpu.make_async_copy(ke_h.at[pl.ds(0, _CH)], kev, sem).wait()

        def compute(j, qv, knv, kev):
            def group(g, c2):
                erow = lax.iota(jnp.int32, 16) + g * 16
                arow = lax.shift_right_logical(erow, 1)
                apar = lax.bitwise_and(erow, 1) * 8

                def head(h, c3):
                    col0 = h * 16
                    acc = jnp.zeros((16,), jnp.float32)
                    for d in range(_DH):
                        ccol = jnp.full((16,), col0 + d, jnp.int32)
                        qq = plsc.load_gather(qv, [erow, ccol])
                        kk = (plsc.load_gather(knv, [erow, ccol]) +
                              plsc.load_gather(kev, [erow, ccol]))
                        acc = acc + qq * kk
                    plsc.store_scatter(av, [arow, apar + h], acc * 0.25)
                    return c3

                lax.fori_loop(0, _H, head, c2)
                return c2

            lax.fori_loop(0, _CH // 16, group, 0)
            pltpu.sync_copy(av, al_h.at[pl.ds((w * epw + j * _CH) // 2, _CH // 2)])

        for b in range(2):
            qv, knv, kev, sem = bufs[b]
            issue(b, qv, knv, kev, sem)

        def outer(i, carry):
            for b in range(2):
                j = 2 * i + b
                qv, knv, kev, sem = bufs[b]

                @pl.when(j < kch)
                def _():
                    drain(qv, knv, kev, sem)
                    compute(j, qv, knv, kev)

                    @pl.when(j + 2 < kch)
                    def _():
                        issue(j + 2, qv, knv, kev, sem)
            return carry

        lax.fori_loop(0, (kch + 1) // 2, outer, 0)

    f = pl.kernel(
        body,
        out_type=jax.ShapeDtypeStruct((e // 2, 16), jnp.float32),
        mesh=_mesh(),
        compiler_params=pltpu.CompilerParams(use_tc_tiling_on_sc=False, needs_layout_passes=False),
        scratch_types=[
            pltpu.VMEM((kch, _CH), jnp.int32),
            pltpu.VMEM((kch, _CH), jnp.int32),
            pltpu.VMEM((_CH, _DP), jnp.float32),
            pltpu.VMEM((_CH, _DP), jnp.float32),
            pltpu.VMEM((_CH, _DP), jnp.float32),
            pltpu.VMEM((_CH, _DP), jnp.float32),
            pltpu.VMEM((_CH, _DP), jnp.float32),
            pltpu.VMEM((_CH, _DP), jnp.float32),
            pltpu.VMEM((_CH // 2, 16), jnp.float32),
            pltpu.SemaphoreType.DMA,
            pltpu.SemaphoreType.DMA,
        ],
    )
    return f(qt, knt, keh, src3, dst3)


_CHB = 40


def _sc_pass_b(vt, veh, al2, gm16, zm, zd, src3, dst3, n):
    kch = src3.shape[1]
    epw = kch * _CHB
    nrs = n // _NS

    def body(vt_h, ve_h, al_h, gm_h, zm_h, zd_h, src_h, dst_h, om_h, od_h,
             srcv, dstv, vv, vev, av, wb, msgb, gmv, accm, accd, sem1):
        c = lax.axis_index("c")
        s = lax.axis_index("s")
        w = s * _NC + c
        pltpu.sync_copy(src_h.at[w], srcv)
        pltpu.sync_copy(dst_h.at[w], dstv)
        pltpu.sync_copy(gm_h, gmv)
        r0 = s * nrs
        pltpu.sync_copy(zm_h.at[pl.ds(r0, nrs)], accm.at[pl.ds(r0, nrs)])
        pltpu.sync_copy(zd_h.at[pl.ds(r0, nrs)], accd.at[pl.ds(r0, nrs)])

        def zrow(r, c2):
            wb[r, :] = jnp.zeros((16,), jnp.float32)
            return c2

        lax.fori_loop(0, _CHB, zrow, 0)
        plsc.subcore_barrier()

        i16 = lax.iota(jnp.int32, 16)
        half = lax.shift_right_logical(i16, 3)
        h8 = lax.bitwise_and(i16, 7)

        def chunk(j, carry):
            cv = pltpu.async_copy(vt_h.at[srcv.at[j]], vv, sem1)
            pltpu.sync_copy(ve_h.at[pl.ds(w * epw + j * _CHB, _CHB)], vev)
            pltpu.sync_copy(al_h.at[pl.ds((w * epw + j * _CHB) // 2, _CHB // 2)], av)
            gmr = gmv[...]

            def wrow(r, c2):
                ww = jnp.exp(av[r, :] - gmr)
                rows = r * 2 + half
                plsc.store_scatter(wb, [rows, h8], ww)
                return c2

            lax.fori_loop(0, _CHB // 2, wrow, 0)
            cv.wait()

            def edge(ei, c2):
                wv = wb[ei, :]
                for h in range(_H):
                    wsc = wv[h]
                    mv = (vv[ei, pl.ds(h * 16, 16)] +
                          vev[ei, pl.ds(h * 16, 16)]) * wsc
                    msgb[ei, pl.ds(h * 16, 16)] = mv
                return c2

            lax.fori_loop(0, _CHB, edge, 0)
            pltpu.sync_copy(msgb, accm.at[dstv.at[j]], add=True)
            pltpu.sync_copy(wb, accd.at[dstv.at[j]], add=True)
            return carry

        lax.fori_loop(0, kch, chunk, 0)
        plsc.subcore_barrier()
        pltpu.sync_copy(accm.at[pl.ds(r0, nrs)], om_h.at[c, pl.ds(r0, nrs)])
        pltpu.sync_copy(accd.at[pl.ds(r0, nrs)], od_h.at[c, pl.ds(r0, nrs)])

    f = pl.kernel(
        body,
        out_type=[jax.ShapeDtypeStruct((_NC, n, _D), jnp.float32),
                  jax.ShapeDtypeStruct((_NC, n, 16), jnp.float32)],
        mesh=_mesh(),
        compiler_params=pltpu.CompilerParams(use_tc_tiling_on_sc=False, needs_layout_passes=False),
        scratch_types=[
            pltpu.VMEM((kch, _CHB), jnp.int32),
            pltpu.VMEM((kch, _CHB), jnp.int32),
            pltpu.VMEM((_CHB, _D), jnp.float32),
            pltpu.VMEM((_CHB, _D), jnp.float32),
            pltpu.VMEM((_CHB // 2, 16), jnp.float32),
            pltpu.VMEM((_CHB, 16), jnp.float32),
            pltpu.VMEM((_CHB, _D), jnp.float32),
            pltpu.VMEM((16,), jnp.float32),
            pltpu.VMEM_SHARED((n, _D), jnp.float32),
            pltpu.VMEM_SHARED((n, 16), jnp.float32),
            pltpu.SemaphoreType.DMA,
        ],
    )
    return f(vt, veh, al2, gm16, zm, zd, src3, dst3)


def _final_stage_kernel(x_ref, g_ref, b_ref, w_ref, wb_ref, o_ref):
    x = x_ref[:]
    m = x.mean(-1, keepdims=True)
    v = ((x - m) ** 2).mean(-1, keepdims=True)
    xn = (x - m) * lax.rsqrt(v + _EPS) * g_ref[:] + b_ref[:]
    o_ref[:] = jnp.dot(xn, w_ref[:], preferred_element_type=jnp.float32) + wb_ref[:]


def _final_stage(x, norm, proj):
    n = x.shape[0]
    br = 1000
    return pl.pallas_call(
        _final_stage_kernel,
        grid=(n // br,),
        in_specs=[
            pl.BlockSpec((br, _D), lambda i: (i, 0)),
            pl.BlockSpec((_D,), lambda i: (0,)),
            pl.BlockSpec((_D,), lambda i: (0,)),
            pl.BlockSpec((_D, _MODES * _D), lambda i: (0, 0)),
            pl.BlockSpec((_MODES * _D,), lambda i: (0,)),
        ],
        out_specs=pl.BlockSpec((br, _MODES * _D), lambda i: (i, 0)),
        out_shape=jax.ShapeDtypeStruct((n, _MODES * _D), jnp.float32),
    )(x, norm["g"], norm["b"], proj["w"], proj["b"])


def _ln_block(x, g, b):
    m = x.mean(-1, keepdims=True)
    v = ((x - m) ** 2).mean(-1, keepdims=True)
    return (x - m) * lax.rsqrt(v + _EPS) * g + b


def _mm(x, w, b):
    return jnp.dot(x, w, preferred_element_type=jnp.float32) + b


def _edge_dense_kernel(sf_ref, df_ref, *refs):
    w10, b10, g10, bb10, w20, b20, w11, b11, g11, bb11, w21, b21, \
        ag1g, ag1b, agw, agb, ag2g, ag2b = refs[:18]
    wrefs = refs[18:24]
    brefs = refs[24:30]
    orefs = refs[30:]
    sf = sf_ref[:]
    df = df_ref[:]
    rel0 = sf[:, 0:1] - df[:, 0:1]
    rel1 = sf[:, 1:2] - df[:, 1:2]
    rp0 = rel0 * df[:, 4:5] + rel1 * df[:, 6:7]
    rp1 = rel0 * df[:, 5:6] + rel1 * df[:, 7:8]
    tf0 = sf[:, 2:3] * df[:, 2:3] + sf[:, 3:4] * df[:, 3:4]
    tf1 = sf[:, 3:4] * df[:, 2:3] - sf[:, 2:3] * df[:, 3:4]
    h0 = jax.nn.relu(_ln_block(rp0 * w10[0:1, :] + rp1 * w10[1:2, :] + b10[:],
                               g10[:], bb10[:]))
    h1 = jax.nn.relu(_ln_block(tf0 * w11[0:1, :] + tf1 * w11[1:2, :] + b11[:],
                               g11[:], bb11[:]))
    o = _mm(h0, w20[:], b20[:]) + _mm(h1, w21[:], b21[:])
    o = jax.nn.relu(_ln_block(o, ag1g[:], ag1b[:]))
    o = _mm(o, agw[:], agb[:])
    ea = _ln_block(o, ag2g[:], ag2b[:])
    zpad = jnp.zeros((ea.shape[0], _DP - _D), jnp.float32)
    for k in range(6):
        o = _mm(ea, wrefs[k][:], brefs[k][:])
        if k % 2 == 0:
            orefs[k][:, :_D] = o
            orefs[k][:, _D:] = zpad
        else:
            orefs[k][:] = o


def _edge_dense(sf, df, rel_p, layers):
    e = sf.shape[0]
    be = 1000
    vecs = []
    mats = []

    def vspec():
        return pl.BlockSpec((_D,), lambda i: (0,))

    br0, br1 = rel_p["branches"]
    w10 = jnp.zeros((8, _D), jnp.float32).at[:2].set(br0["lin1"]["w"])
    w11 = jnp.zeros((8, _D), jnp.float32).at[:2].set(br1["lin1"]["w"])
    args = [w10, br0["lin1"]["b"], br0["ln"]["g"], br0["ln"]["b"],
            br0["lin2"]["w"], br0["lin2"]["b"],
            w11, br1["lin1"]["b"], br1["ln"]["g"], br1["ln"]["b"],
            br1["lin2"]["w"], br1["lin2"]["b"],
            rel_p["aggr_ln1"]["g"], rel_p["aggr_ln1"]["b"],
            rel_p["aggr_lin"]["w"], rel_p["aggr_lin"]["b"],
            rel_p["aggr_ln2"]["g"], rel_p["aggr_ln2"]["b"]]
    args += [lp[k]["w"] for lp in layers for k in ("lin_k_edge", "lin_v_edge")]
    args += [lp[k]["b"] for lp in layers for k in ("lin_k_edge", "lin_v_edge")]
    wspecs = []
    for a in args:
        if a.ndim == 1:
            wspecs.append(vspec())
        else:
            sh = a.shape
            wspecs.append(pl.BlockSpec(sh, lambda i: (0, 0)))
    outs = pl.pallas_call(
        _edge_dense_kernel,
        grid=(e // be,),
        in_specs=[pl.BlockSpec((be, 16), lambda i: (i, 0)),
                  pl.BlockSpec((be, 16), lambda i: (i, 0))] + wspecs,
        out_specs=[pl.BlockSpec((be, _DP if k % 2 == 0 else _D),
                                lambda i: (i, 0)) for k in range(6)],
        out_shape=[jax.ShapeDtypeStruct((e, _DP if k % 2 == 0 else _D),
                                        jnp.float32) for k in range(6)],
    )(sf, df, *args)
    return outs


def _node_pre_kernel(x_ref, g_ref, b_ref, wq, bq, wk, bk, wv, bv,
                     h_ref, q_ref, k_ref, v_ref):
    h = _ln_block(x_ref[:], g_ref[:], b_ref[:])
    h_ref[:] = h
    zpad = jnp.zeros((h.shape[0], _DP - _D), jnp.float32)
    q_ref[:, :_D] = _mm(h, wq[:], bq[:])
    q_ref[:, _D:] = zpad
    k_ref[:, :_D] = _mm(h, wk[:], bk[:])
    k_ref[:, _D:] = zpad
    v_ref[:] = _mm(h, wv[:], bv[:])


def _node_pre(x, lp):
    n = x.shape[0]
    bn = 1000
    mspec = pl.BlockSpec((_D, _D), lambda i: (0, 0))
    vspec = pl.BlockSpec((_D,), lambda i: (0,))
    rspec = pl.BlockSpec((bn, _D), lambda i: (i, 0))
    return pl.pallas_call(
        _node_pre_kernel,
        grid=(n // bn,),
        in_specs=[rspec, vspec, vspec, mspec, vspec, mspec, vspec, mspec, vspec],
        out_specs=[rspec,
                   pl.BlockSpec((bn, _DP), lambda i: (i, 0)),
                   pl.BlockSpec((bn, _DP), lambda i: (i, 0)),
                   rspec],
        out_shape=[jax.ShapeDtypeStruct((n, _D), jnp.float32),
                   jax.ShapeDtypeStruct((n, _DP), jnp.float32),
                   jax.ShapeDtypeStruct((n, _DP), jnp.float32),
                   jax.ShapeDtypeStruct((n, _D), jnp.float32)],
    )(x, lp["norm1"]["g"], lp["norm1"]["b"],
      lp["lin_q_node"]["w"], lp["lin_q_node"]["b"],
      lp["lin_k_node"]["w"], lp["lin_k_node"]["b"],
      lp["lin_v_node"]["w"], lp["lin_v_node"]["b"])


def _node_post_kernel(om0, om1, od0, od1, rmat, h_ref, x_ref,
                      wih, bih, whh, bhh, wsf, bsf, wop, bop,
                      n2g, n2b, w1, b1, w2, b2, o_ref):
    den = jnp.dot(od0[:] + od1[:], rmat[:],
                  preferred_element_type=jnp.float32) + 1e-16
    agg = (om0[:] + om1[:]) / den
    h = h_ref[:]
    gate = jax.nn.sigmoid(_mm(agg, wih[:], bih[:]) + _mm(h, whh[:], bhh[:]))
    upd = agg + gate * (_mm(h, wsf[:], bsf[:]) - agg)
    x2 = x_ref[:] + _mm(upd, wop[:], bop[:])
    h2 = _ln_block(x2, n2g[:], n2b[:])
    ff = _mm(jax.nn.relu(_mm(h2, w1[:], b1[:])), w2[:], b2[:])
    o_ref[:] = x2 + ff


def _node_post(om, od, rmat, h, x, lp):
    n = x.shape[0]
    bn = 1000
    rspec = pl.BlockSpec((bn, _D), lambda i: (i, 0))
    dspec = pl.BlockSpec((bn, 16), lambda i: (i, 0))
    vspec = pl.BlockSpec((_D,), lambda i: (0,))

    def mspec(a, b):
        return pl.BlockSpec((a, b), lambda i: (0, 0))

    return pl.pallas_call(
        _node_post_kernel,
        grid=(n // bn,),
        in_specs=[rspec, rspec, dspec, dspec, mspec(16, _D), rspec, rspec,
                  mspec(_D, _D), vspec, mspec(_D, _D), vspec,
                  mspec(_D, _D), vspec, mspec(_D, _D), vspec,
                  vspec, vspec,
                  mspec(_D, 4 * _D), pl.BlockSpec((4 * _D,), lambda i: (0,)),
                  mspec(4 * _D, _D), vspec],
        out_specs=rspec,
        out_shape=jax.ShapeDtypeStruct((n, _D), jnp.float32),
    )(om[0], om[1], od[0], od[1], rmat, h, x,
      lp["lin_ih"]["w"], lp["lin_ih"]["b"],
      lp["lin_hh"]["w"], lp["lin_hh"]["b"],
      lp["lin_self"]["w"], lp["lin_self"]["b"],
      lp["out_proj"]["w"], lp["out_proj"]["b"],
      lp["norm2"]["g"], lp["norm2"]["b"],
      lp["mlp_lin1"]["w"], lp["mlp_lin1"]["b"],
      lp["mlp_lin2"]["w"], lp["mlp_lin2"]["b"])


def _alpha_max_kernel(a_ref, o_ref):
    @pl.when(pl.program_id(0) == 0)
    def _init():
        o_ref[:] = jnp.full((8, 16), -jnp.inf, jnp.float32)

    blk = a_ref[:].reshape(-1, 8, 16).max(axis=0)
    o_ref[:] = jnp.maximum(o_ref[:], blk)


def _alpha_max(al2):
    m = al2.shape[0]
    bm = 1000
    out = pl.pallas_call(
        _alpha_max_kernel,
        grid=(m // bm,),
        in_specs=[pl.BlockSpec((bm, 16), lambda i: (i, 0))],
        out_specs=pl.BlockSpec((8, 16), lambda i: (0, 0)),
        out_shape=jax.ShapeDtypeStruct((8, 16), jnp.float32),
    )(al2)
    return out.max(axis=0)


def _layer(lp, x, src3, dst3, src3b, dst3b, ke, ve, rmat, zm, zd, n):
    h, q, kn, vn = _node_pre(x, lp)
    al2 = _sc_pass_a(q, kn, ke, src3, dst3)
    m16 = _alpha_max(al2)
    gm8 = jnp.maximum(m16[:8], m16[8:])
    gm16 = jnp.concatenate([gm8, gm8])
    om, od = _sc_pass_b(vn, ve, al2, gm16, zm, zd, src3b, dst3b, n)
    return _node_post(om, od, rmat, h, x, lp)


@jax.jit
def _run(local_embed, edge_index, positions, rotate_mat, rotate_angles, params):
    n, d = local_embed.shape
    t = positions.shape[1]
    e = edge_index.shape[1]
    kch = e // (_NW * _CH)
    src3 = edge_index[0].reshape(_NW, kch, _CH)
    dst3 = edge_index[1].reshape(_NW, kch, _CH)
    kchb = e // (_NW * _CHB)
    src3b = edge_index[0].reshape(_NW, kchb, _CHB)
    dst3b = edge_index[1].reshape(_NW, kchb, _CHB)

    p19 = positions[:, t - 1]
    ca = jnp.cos(rotate_angles)[:, None]
    sa = jnp.sin(rotate_angles)[:, None]
    nf = jnp.concatenate(
        [p19, ca, sa, rotate_mat.reshape(n, 4), jnp.zeros((n, 8), jnp.float32)],
        axis=1)
    sf, df = _sc_edge_gather(nf, src3, dst3)

    keve = _edge_dense(sf, df, params["rel_embed"], params["layers"])

    rmat = jnp.concatenate(
        [jnp.repeat(jnp.eye(_H, dtype=jnp.float32), _DH, axis=1),
         jnp.zeros((_H, _D), jnp.float32)], axis=0)
    zm = jnp.zeros((n, _D), jnp.float32)
    zd = jnp.zeros((n, 16), jnp.float32)
    x = local_embed
    for li, lp in enumerate(params["layers"]):
        x = _layer(lp, x, src3, dst3, src3b, dst3b, keve[2 * li],
                   keve[2 * li + 1], rmat, zm, zd, n)
    out = _final_stage(x, params["norm"], params["multihead_proj"])
    return jnp.transpose(out.reshape(n, _MODES, d), (1, 0, 2))


def kernel(local_embed, edge_index, positions, rotate_mat, rotate_angles, padding_mask, params):
    return _run(local_embed, edge_index, positions, rotate_mat, rotate_angles, params)


# speedup vs baseline: 1.5161x; 1.0000x over previous
import functools
import numpy as np
import jax
import jax.numpy as jnp
from jax import lax
from jax.experimental import pallas as pl
from jax.experimental.pallas import tpu as pltpu
from jax.experimental.pallas import tpu_sc as plsc

_D = 128
_H = 8
_DH = 16
_MODES = 6
_EPS = 1e-5
_DP = 144
_NC = 2
_NS = 16
_NW = _NC * _NS
_CH = 80


def _ln(p, x):
    m = x.mean(-1, keepdims=True)
    v = ((x - m) ** 2).mean(-1, keepdims=True)
    return (x - m) / jnp.sqrt(v + _EPS) * p["g"] + p["b"]


def _lin(p, x):
    return x @ p["w"] + p["b"]


def _mesh():
    return plsc.VectorSubcoreMesh(core_axis_name="c", subcore_axis_name="s")


def _wid():
    return lax.axis_index("s") * _NC + lax.axis_index("c")


def _sc_edge_gather(nf, src3, dst3):
    kch = src3.shape[1]
    epw = kch * _CH
    e = _NW * epw

    def body(nf_h, src_h, dst_h, sf_h, df_h, srcv, dstv, bs, bd, sem1, sem2):
        w = _wid()
        pltpu.sync_copy(src_h.at[w], srcv)
        pltpu.sync_copy(dst_h.at[w], dstv)

        def chunk(j, carry):
            cs = pltpu.async_copy(nf_h.at[srcv.at[j]], bs, sem1)
            cd = pltpu.async_copy(nf_h.at[dstv.at[j]], bd, sem2)
            cs.wait()
            cd.wait()
            r0 = w * epw + j * _CH
            pltpu.sync_copy(bs, sf_h.at[pl.ds(r0, _CH)])
            pltpu.sync_copy(bd, df_h.at[pl.ds(r0, _CH)])
            return carry

        lax.fori_loop(0, kch, chunk, 0)

    f = pl.kernel(
        body,
        out_type=[jax.ShapeDtypeStruct((e, 16), jnp.float32),
                  jax.ShapeDtypeStruct((e, 16), jnp.float32)],
        mesh=_mesh(),
        compiler_params=pltpu.CompilerParams(use_tc_tiling_on_sc=False, needs_layout_passes=False),
        scratch_types=[
            pltpu.VMEM((kch, _CH), jnp.int32),
            pltpu.VMEM((kch, _CH), jnp.int32),
            pltpu.VMEM((_CH, 16), jnp.float32),
            pltpu.VMEM((_CH, 16), jnp.float32),
            pltpu.SemaphoreType.DMA,
            pltpu.SemaphoreType.DMA,
        ],
    )
    return f(nf, src3, dst3)


def _sc_pass_a(qt, knt, keh, src3, dst3):
    kch = src3.shape[1]
    epw = kch * _CH
    e = _NW * epw

    def body(qt_h, knt_h, ke_h, src_h, dst_h, al_h, srcv, dstv,
             qv0, knv0, kev0, qv1, knv1, kev1, av, sem0, sem1):
        w = _wid()
        pltpu.sync_copy(src_h.at[w], srcv)
        pltpu.sync_copy(dst_h.at[w], dstv)
        bufs = ((qv0, knv0, kev0, sem0), (qv1, knv1, kev1, sem1))

        def issue(j, qv, knv, kev, sem):
            pltpu.async_copy(qt_h.at[dstv.at[j]], qv, sem)
            pltpu.async_copy(knt_h.at[srcv.at[j]], knv, sem)
            pltpu.async_copy(ke_h.at[pl.ds(w * epw + j * _CH, _CH)], kev, sem)

        def drain(j, qv, knv, kev, sem):
            pltpu.make_async_copy(qt_h.at[dstv.at[j]], qv, sem).wait()
            pltpu.make_async_copy(knt_h.at[srcv.at[j]], knv, sem).wait()
            pltpu.make_async_copy(
                ke_h.at[pl.ds(w * epw + j * _CH, _CH)], kev, sem).wait()

        def compute(j, qv, knv, kev):
            def group(g, c2):
                erow = lax.iota(jnp.int32, 16) + g * 16
                arow = lax.shift_right_logical(erow, 1)
                apar = lax.bitwise_and(erow, 1) * 8

                def head(h, c3):
                    col0 = h * 16
                    acc = jnp.zeros((16,), jnp.float32)
                    for d in range(_DH):
                        ccol = jnp.full((16,), col0 + d, jnp.int32)
                        qq = plsc.load_gather(qv, [erow, ccol])
                        kk = (plsc.load_gather(knv, [erow, ccol]) +
                              plsc.load_gather(kev, [erow, ccol]))
                        acc = acc + qq * kk
                    plsc.store_scatter(av, [arow, apar + h], acc * 0.25)
                    return c3

                lax.fori_loop(0, _H, head, c2)
                return c2

            lax.fori_loop(0, _CH // 16, group, 0)
            pltpu.sync_copy(av, al_h.at[pl.ds((w * epw + j * _CH) // 2, _CH // 2)])

        for b in range(2):
            qv, knv, kev, sem = bufs[b]
            issue(b, qv, knv, kev, sem)

        def outer(i, carry):
            for b in range(2):
                j = 2 * i + b
                qv, knv, kev, sem = bufs[b]

                @pl.when(j < kch)
                def _():
                    drain(j, qv, knv, kev, sem)
                    compute(j, qv, knv, kev)

                    @pl.when(j + 2 < kch)
                    def _():
                        issue(j + 2, qv, knv, kev, sem)
            return carry

        lax.fori_loop(0, (kch + 1) // 2, outer, 0)

    f = pl.kernel(
        body,
        out_type=jax.ShapeDtypeStruct((e // 2, 16), jnp.float32),
        mesh=_mesh(),
        compiler_params=pltpu.CompilerParams(use_tc_tiling_on_sc=False, needs_layout_passes=False),
        scratch_types=[
            pltpu.VMEM((kch, _CH), jnp.int32),
            pltpu.VMEM((kch, _CH), jnp.int32),
            pltpu.VMEM((_CH, _DP), jnp.float32),
            pltpu.VMEM((_CH, _DP), jnp.float32),
            pltpu.VMEM((_CH, _DP), jnp.float32),
            pltpu.VMEM((_CH, _DP), jnp.float32),
            pltpu.VMEM((_CH, _DP), jnp.float32),
            pltpu.VMEM((_CH, _DP), jnp.float32),
            pltpu.VMEM((_CH // 2, 16), jnp.float32),
            pltpu.SemaphoreType.DMA,
            pltpu.SemaphoreType.DMA,
        ],
    )
    return f(qt, knt, keh, src3, dst3)


_CHB = 40


def _sc_pass_b(vt, veh, al2, gm16, zm, zd, src3, dst3, n):
    kch = src3.shape[1]
    epw = kch * _CHB
    nrs = n // _NS

    def body(vt_h, ve_h, al_h, gm_h, zm_h, zd_h, src_h, dst_h, om_h, od_h,
             srcv, dstv, vv, vev, av, wb, msgb, gmv, accm, accd, sem1):
        c = lax.axis_index("c")
        s = lax.axis_index("s")
        w = s * _NC + c
        pltpu.sync_copy(src_h.at[w], srcv)
        pltpu.sync_copy(dst_h.at[w], dstv)
        pltpu.sync_copy(gm_h, gmv)
        r0 = s * nrs
        pltpu.sync_copy(zm_h.at[pl.ds(r0, nrs)], accm.at[pl.ds(r0, nrs)])
        pltpu.sync_copy(zd_h.at[pl.ds(r0, nrs)], accd.at[pl.ds(r0, nrs)])

        def zrow(r, c2):
            wb[r, :] = jnp.zeros((16,), jnp.float32)
            return c2

        lax.fori_loop(0, _CHB, zrow, 0)
        plsc.subcore_barrier()

        i16 = lax.iota(jnp.int32, 16)
        half = lax.shift_right_logical(i16, 3)
        h8 = lax.bitwise_and(i16, 7)

        def chunk(j, carry):
            cv = pltpu.async_copy(vt_h.at[srcv.at[j]], vv, sem1)
            pltpu.sync_copy(ve_h.at[pl.ds(w * epw + j * _CHB, _CHB)], vev)
            pltpu.sync_copy(al_h.at[pl.ds((w * epw + j * _CHB) // 2, _CHB // 2)], av)
            gmr = gmv[...]

            def wrow(r, c2):
                ww = jnp.exp(av[r, :] - gmr)
                rows = r * 2 + half
                plsc.store_scatter(wb, [rows, h8], ww)
                return c2

            lax.fori_loop(0, _CHB // 2, wrow, 0)
            cv.wait()

            def edge(ei, c2):
                wv = wb[ei, :]
                for h in range(_H):
                    wsc = wv[h]
                    mv = (vv[ei, pl.ds(h * 16, 16)] +
                          vev[ei, pl.ds(h * 16, 16)]) * wsc
                    msgb[ei, pl.ds(h * 16, 16)] = mv
                return c2

            lax.fori_loop(0, _CHB, edge, 0)
            pltpu.sync_copy(msgb, accm.at[dstv.at[j]], add=True)
            pltpu.sync_copy(wb, accd.at[dstv.at[j]], add=True)
            return carry

        lax.fori_loop(0, kch, chunk, 0)
        plsc.subcore_barrier()
        pltpu.sync_copy(accm.at[pl.ds(r0, nrs)], om_h.at[c, pl.ds(r0, nrs)])
        pltpu.sync_copy(accd.at[pl.ds(r0, nrs)], od_h.at[c, pl.ds(r0, nrs)])

    f = pl.kernel(
        body,
        out_type=[jax.ShapeDtypeStruct((_NC, n, _D), jnp.float32),
                  jax.ShapeDtypeStruct((_NC, n, 16), jnp.float32)],
        mesh=_mesh(),
        compiler_params=pltpu.CompilerParams(use_tc_tiling_on_sc=False, needs_layout_passes=False),
        scratch_types=[
            pltpu.VMEM((kch, _CHB), jnp.int32),
            pltpu.VMEM((kch, _CHB), jnp.int32),
            pltpu.VMEM((_CHB, _D), jnp.float32),
            pltpu.VMEM((_CHB, _D), jnp.float32),
            pltpu.VMEM((_CHB // 2, 16), jnp.float32),
            pltpu.VMEM((_CHB, 16), jnp.float32),
            pltpu.VMEM((_CHB, _D), jnp.float32),
            pltpu.VMEM((16,), jnp.float32),
            pltpu.VMEM_SHARED((n, _D), jnp.float32),
            pltpu.VMEM_SHARED((n, 16), jnp.float32),
            pltpu.SemaphoreType.DMA,
        ],
    )
    return f(vt, veh, al2, gm16, zm, zd, src3, dst3)


def _final_stage_kernel(x_ref, g_ref, b_ref, w_ref, wb_ref, o_ref):
    x = x_ref[:]
    m = x.mean(-1, keepdims=True)
    v = ((x - m) ** 2).mean(-1, keepdims=True)
    xn = (x - m) * lax.rsqrt(v + _EPS) * g_ref[:] + b_ref[:]
    o_ref[:] = jnp.dot(xn, w_ref[:], preferred_element_type=jnp.float32) + wb_ref[:]


def _final_stage(x, norm, proj):
    n = x.shape[0]
    br = 1000
    return pl.pallas_call(
        _final_stage_kernel,
        grid=(n // br,),
        in_specs=[
            pl.BlockSpec((br, _D), lambda i: (i, 0)),
            pl.BlockSpec((_D,), lambda i: (0,)),
            pl.BlockSpec((_D,), lambda i: (0,)),
            pl.BlockSpec((_D, _MODES * _D), lambda i: (0, 0)),
            pl.BlockSpec((_MODES * _D,), lambda i: (0,)),
        ],
        out_specs=pl.BlockSpec((br, _MODES * _D), lambda i: (i, 0)),
        out_shape=jax.ShapeDtypeStruct((n, _MODES * _D), jnp.float32),
    )(x, norm["g"], norm["b"], proj["w"], proj["b"])


def _ln_block(x, g, b):
    m = x.mean(-1, keepdims=True)
    v = ((x - m) ** 2).mean(-1, keepdims=True)
    return (x - m) * lax.rsqrt(v + _EPS) * g + b


def _mm(x, w, b):
    return jnp.dot(x, w, preferred_element_type=jnp.float32) + b


def _edge_dense_kernel(sf_ref, df_ref, *refs):
    w10, b10, g10, bb10, w20, b20, w11, b11, g11, bb11, w21, b21, \
        ag1g, ag1b, agw, agb, ag2g, ag2b = refs[:18]
    wrefs = refs[18:24]
    brefs = refs[24:30]
    orefs = refs[30:]
    sf = sf_ref[:]
    df = df_ref[:]
    rel0 = sf[:, 0:1] - df[:, 0:1]
    rel1 = sf[:, 1:2] - df[:, 1:2]
    rp0 = rel0 * df[:, 4:5] + rel1 * df[:, 6:7]
    rp1 = rel0 * df[:, 5:6] + rel1 * df[:, 7:8]
    tf0 = sf[:, 2:3] * df[:, 2:3] + sf[:, 3:4] * df[:, 3:4]
    tf1 = sf[:, 3:4] * df[:, 2:3] - sf[:, 2:3] * df[:, 3:4]
    h0 = jax.nn.relu(_ln_block(rp0 * w10[0:1, :] + rp1 * w10[1:2, :] + b10[:],
                               g10[:], bb10[:]))
    h1 = jax.nn.relu(_ln_block(tf0 * w11[0:1, :] + tf1 * w11[1:2, :] + b11[:],
                               g11[:], bb11[:]))
    o = _mm(h0, w20[:], b20[:]) + _mm(h1, w21[:], b21[:])
    o = jax.nn.relu(_ln_block(o, ag1g[:], ag1b[:]))
    o = _mm(o, agw[:], agb[:])
    ea = _ln_block(o, ag2g[:], ag2b[:])
    zpad = jnp.zeros((ea.shape[0], _DP - _D), jnp.float32)
    for k in range(6):
        o = _mm(ea, wrefs[k][:], brefs[k][:])
        if k % 2 == 0:
            orefs[k][:, :_D] = o
            orefs[k][:, _D:] = zpad
        else:
            orefs[k][:] = o


def _edge_dense(sf, df, rel_p, layers):
    e = sf.shape[0]
    be = 1000
    vecs = []
    mats = []

    def vspec():
        return pl.BlockSpec((_D,), lambda i: (0,))

    br0, br1 = rel_p["branches"]
    w10 = jnp.zeros((8, _D), jnp.float32).at[:2].set(br0["lin1"]["w"])
    w11 = jnp.zeros((8, _D), jnp.float32).at[:2].set(br1["lin1"]["w"])
    args = [w10, br0["lin1"]["b"], br0["ln"]["g"], br0["ln"]["b"],
            br0["lin2"]["w"], br0["lin2"]["b"],
            w11, br1["lin1"]["b"], br1["ln"]["g"], br1["ln"]["b"],
            br1["lin2"]["w"], br1["lin2"]["b"],
            rel_p["aggr_ln1"]["g"], rel_p["aggr_ln1"]["b"],
            rel_p["aggr_lin"]["w"], rel_p["aggr_lin"]["b"],
            rel_p["aggr_ln2"]["g"], rel_p["aggr_ln2"]["b"]]
    args += [lp[k]["w"] for lp in layers for k in ("lin_k_edge", "lin_v_edge")]
    args += [lp[k]["b"] for lp in layers for k in ("lin_k_edge", "lin_v_edge")]
    wspecs = []
    for a in args:
        if a.ndim == 1:
            wspecs.append(vspec())
        else:
            sh = a.shape
            wspecs.append(pl.BlockSpec(sh, lambda i: (0, 0)))
    outs = pl.pallas_call(
        _edge_dense_kernel,
        grid=(e // be,),
        in_specs=[pl.BlockSpec((be, 16), lambda i: (i, 0)),
                  pl.BlockSpec((be, 16), lambda i: (i, 0))] + wspecs,
        out_specs=[pl.BlockSpec((be, _DP if k % 2 == 0 else _D),
                                lambda i: (i, 0)) for k in range(6)],
        out_shape=[jax.ShapeDtypeStruct((e, _DP if k % 2 == 0 else _D),
                                        jnp.float32) for k in range(6)],
    )(sf, df, *args)
    return outs


def _node_pre_kernel(x_ref, g_ref, b_ref, wq, bq, wk, bk, wv, bv,
                     h_ref, q_ref, k_ref, v_ref):
    h = _ln_block(x_ref[:], g_ref[:], b_ref[:])
    h_ref[:] = h
    zpad = jnp.zeros((h.shape[0], _DP - _D), jnp.float32)
    q_ref[:, :_D] = _mm(h, wq[:], bq[:])
    q_ref[:, _D:] = zpad
    k_ref[:, :_D] = _mm(h, wk[:], bk[:])
    k_ref[:, _D:] = zpad
    v_ref[:] = _mm(h, wv[:], bv[:])


def _node_pre(x, lp):
    n = x.shape[0]
    bn = 1000
    mspec = pl.BlockSpec((_D, _D), lambda i: (0, 0))
    vspec = pl.BlockSpec((_D,), lambda i: (0,))
    rspec = pl.BlockSpec((bn, _D), lambda i: (i, 0))
    return pl.pallas_call(
        _node_pre_kernel,
        grid=(n // bn,),
        in_specs=[rspec, vspec, vspec, mspec, vspec, mspec, vspec, mspec, vspec],
        out_specs=[rspec,
                   pl.BlockSpec((bn, _DP), lambda i: (i, 0)),
                   pl.BlockSpec((bn, _DP), lambda i: (i, 0)),
                   rspec],
        out_shape=[jax.ShapeDtypeStruct((n, _D), jnp.float32),
                   jax.ShapeDtypeStruct((n, _DP), jnp.float32),
                   jax.ShapeDtypeStruct((n, _DP), jnp.float32),
                   jax.ShapeDtypeStruct((n, _D), jnp.float32)],
    )(x, lp["norm1"]["g"], lp["norm1"]["b"],
      lp["lin_q_node"]["w"], lp["lin_q_node"]["b"],
      lp["lin_k_node"]["w"], lp["lin_k_node"]["b"],
      lp["lin_v_node"]["w"], lp["lin_v_node"]["b"])


def _node_post_kernel(om0, om1, od0, od1, rmat, h_ref, x_ref,
                      wih, bih, whh, bhh, wsf, bsf, wop, bop,
                      n2g, n2b, w1, b1, w2, b2, o_ref):
    den = jnp.dot(od0[:] + od1[:], rmat[:],
                  preferred_element_type=jnp.float32) + 1e-16
    agg = (om0[:] + om1[:]) / den
    h = h_ref[:]
    gate = jax.nn.sigmoid(_mm(agg, wih[:], bih[:]) + _mm(h, whh[:], bhh[:]))
    upd = agg + gate * (_mm(h, wsf[:], bsf[:]) - agg)
    x2 = x_ref[:] + _mm(upd, wop[:], bop[:])
    h2 = _ln_block(x2, n2g[:], n2b[:])
    ff = _mm(jax.nn.relu(_mm(h2, w1[:], b1[:])), w2[:], b2[:])
    o_ref[:] = x2 + ff


def _node_post(om, od, rmat, h, x, lp):
    n = x.shape[0]
    bn = 1000
    rspec = pl.BlockSpec((bn, _D), lambda i: (i, 0))
    dspec = pl.BlockSpec((bn, 16), lambda i: (i, 0))
    vspec = pl.BlockSpec((_D,), lambda i: (0,))

    def mspec(a, b):
        return pl.BlockSpec((a, b), lambda i: (0, 0))

    return pl.pallas_call(
        _node_post_kernel,
        grid=(n // bn,),
        in_specs=[rspec, rspec, dspec, dspec, mspec(16, _D), rspec, rspec,
                  mspec(_D, _D), vspec, mspec(_D, _D), vspec,
                  mspec(_D, _D), vspec, mspec(_D, _D), vspec,
                  vspec, vspec,
                  mspec(_D, 4 * _D), pl.BlockSpec((4 * _D,), lambda i: (0,)),
                  mspec(4 * _D, _D), vspec],
        out_specs=rspec,
        out_shape=jax.ShapeDtypeStruct((n, _D), jnp.float32),
    )(om[0], om[1], od[0], od[1], rmat, h, x,
      lp["lin_ih"]["w"], lp["lin_ih"]["b"],
      lp["lin_hh"]["w"], lp["lin_hh"]["b"],
      lp["lin_self"]["w"], lp["lin_self"]["b"],
      lp["out_proj"]["w"], lp["out_proj"]["b"],
      lp["norm2"]["g"], lp["norm2"]["b"],
      lp["mlp_lin1"]["w"], lp["mlp_lin1"]["b"],
      lp["mlp_lin2"]["w"], lp["mlp_lin2"]["b"])


def _alpha_max_kernel(a_ref, o_ref):
    @pl.when(pl.program_id(0) == 0)
    def _init():
        o_ref[:] = jnp.full((8, 16), -jnp.inf, jnp.float32)

    blk = a_ref[:].reshape(-1, 8, 16).max(axis=0)
    o_ref[:] = jnp.maximum(o_ref[:], blk)


def _alpha_max(al2):
    m = al2.shape[0]
    bm = 1000
    out = pl.pallas_call(
        _alpha_max_kernel,
        grid=(m // bm,),
        in_specs=[pl.BlockSpec((bm, 16), lambda i: (i, 0))],
        out_specs=pl.BlockSpec((8, 16), lambda i: (0, 0)),
        out_shape=jax.ShapeDtypeStruct((8, 16), jnp.float32),
    )(al2)
    return out.max(axis=0)


def _layer(lp, x, src3, dst3, src3b, dst3b, ke, ve, rmat, zm, zd, n):
    h, q, kn, vn = _node_pre(x, lp)
    al2 = _sc_pass_a(q, kn, ke, src3, dst3)
    m16 = _alpha_max(al2)
    gm8 = jnp.maximum(m16[:8], m16[8:])
    gm16 = jnp.concatenate([gm8, gm8])
    om, od = _sc_pass_b(vn, ve, al2, gm16, zm, zd, src3b, dst3b, n)
    return _node_post(om, od, rmat, h, x, lp)


@jax.jit
def _run(local_embed, edge_index, positions, rotate_mat, rotate_angles, params):
    n, d = local_embed.shape
    t = positions.shape[1]
    e = edge_index.shape[1]
    kch = e // (_NW * _CH)
    src3 = edge_index[0].reshape(_NW, kch, _CH)
    dst3 = edge_index[1].reshape(_NW, kch, _CH)
    kchb = e // (_NW * _CHB)
    src3b = edge_index[0].reshape(_NW, kchb, _CHB)
    dst3b = edge_index[1].reshape(_NW, kchb, _CHB)

    p19 = positions[:, t - 1]
    ca = jnp.cos(rotate_angles)[:, None]
    sa = jnp.sin(rotate_angles)[:, None]
    nf = jnp.concatenate(
        [p19, ca, sa, rotate_mat.reshape(n, 4), jnp.zeros((n, 8), jnp.float32)],
        axis=1)
    sf, df = _sc_edge_gather(nf, src3, dst3)

    keve = _edge_dense(sf, df, params["rel_embed"], params["layers"])

    rmat = jnp.concatenate(
        [jnp.repeat(jnp.eye(_H, dtype=jnp.float32), _DH, axis=1),
         jnp.zeros((_H, _D), jnp.float32)], axis=0)
    zm = jnp.zeros((n, _D), jnp.float32)
    zd = jnp.zeros((n, 16), jnp.float32)
    x = local_embed
    for li, lp in enumerate(params["layers"]):
        x = _layer(lp, x, src3, dst3, src3b, dst3b, keve[2 * li],
                   keve[2 * li + 1], rmat, zm, zd, n)
    out = _final_stage(x, params["norm"], params["multihead_proj"])
    return jnp.transpose(out.reshape(n, _MODES, d), (1, 0, 2))


def kernel(local_embed, edge_index, positions, rotate_mat, rotate_angles, padding_mask, params):
    return _run(local_embed, edge_index, positions, rotate_mat, rotate_angles, params)
